# baseline (speedup 1.0000x reference)
import functools

import jax
import jax.numpy as jnp
from jax import lax
from jax.experimental import pallas as pl
from jax.experimental.pallas import tpu as pltpu
from jax.experimental.pallas import tpu_sc as plsc

NUM_EXPERTS = 8
TOP_K = 2
HIDDEN = 1024
INTER = 4096
T = 2048

BLK = 576
JBLK = 2048
NJ = INTER // JBLK
NPAD = ((2 * T + NUM_EXPERTS * (BLK - 1)) // BLK + 1) * BLK
NB = NPAD // BLK

NC, NS = 2, 16
NW = NC * NS
A_PER_W = (TOP_K * T) // NW
T_PER_W = T // NW
DCH = 64
CCH = 32


def _cumsum_rows(x):
    n = x.shape[0]
    sh = 1
    while sh < n:
        x = x + jnp.concatenate(
            [jnp.zeros((sh,) + x.shape[1:], x.dtype), x[:-sh]], axis=0)
        sh *= 2
    return x


def _cumsum_lanes(x):
    n = x.shape[1]
    sh = 1
    while sh < n:
        x = x + jnp.concatenate(
            [jnp.zeros((x.shape[0], sh), x.dtype), x[:, :-sh]], axis=1)
        sh *= 2
    return x


def _router_body(x_ref, wr_ref, scale_ref, slot_ref, w_ref, plan_ref):
    x = x_ref[...]
    wr = wr_ref[...]
    logits = jnp.dot(x, wr, preferred_element_type=jnp.float32)
    m = jnp.max(logits, axis=-1, keepdims=True)
    p = jnp.exp(logits - m)
    p = p / jnp.sum(p, axis=-1, keepdims=True)

    e_iota = jax.lax.broadcasted_iota(jnp.int32, (T, NUM_EXPERTS), 1)

    p1 = jnp.max(p, axis=-1, keepdims=True)
    i1 = jnp.min(jnp.where(p == p1, e_iota, NUM_EXPERTS), axis=-1,
                 keepdims=True)
    masked = jnp.where(e_iota == i1, -jnp.inf, p)
    p2 = jnp.max(masked, axis=-1, keepdims=True)
    i2 = jnp.min(jnp.where(masked == p2, e_iota, NUM_EXPERTS), axis=-1,
                 keepdims=True)

    bb = jnp.exp(p2 - p1)
    w1 = 1.0 / (1.0 + bb)
    w2 = bb / (1.0 + bb)

    scale = scale_ref[...]
    s1 = jnp.sum(jnp.where(e_iota == i1, scale, 0.0), axis=-1, keepdims=True)
    s2 = jnp.sum(jnp.where(e_iota == i2, scale, 0.0), axis=-1, keepdims=True)

    oh0 = (e_iota == i1).astype(jnp.int32)
    oh1 = (e_iota == i2).astype(jnp.int32)
    c0 = _cumsum_rows(oh0)
    c1 = _cumsum_rows(oh1)
    pos0 = c0 - oh0
    pos1 = c1 - oh1
    count0 = c0[T - 1:T, :]
    counts = count0 + c1[T - 1:T, :]
    counts_pad = ((counts + (BLK - 1)) // BLK) * BLK
    base = _cumsum_lanes(counts_pad) - counts_pad

    dest0 = base + pos0
    dest1 = base + count0 + pos1
    slot0 = jnp.sum(oh0 * dest0, axis=-1)
    slot1 = jnp.sum(oh1 * dest1, axis=-1)
    slot_ref[0:1, :] = slot0.reshape(1, T)
    slot_ref[1:2, :] = slot1.reshape(1, T)
    w_ref[0:1, :] = (w1 * s1).reshape(1, T)
    w_ref[1:2, :] = (w2 * s2).reshape(1, T)

    ends = base + counts_pad
    b_iota = jax.lax.broadcasted_iota(jnp.int32, (NB, NUM_EXPERTS), 0) * BLK
    be = jnp.sum((b_iota >= ends).astype(jnp.int32), axis=-1)
    be = jnp.minimum(be, NUM_EXPERTS - 1)
    na = jnp.sum(counts_pad, axis=-1, keepdims=True) // BLK
    plan_ref[...] = jnp.concatenate([be.reshape(1, NB), na], axis=1)


def _router(x, Wr, expert_scale):
    return pl.pallas_call(
        _router_body,
        out_shape=(
            jax.ShapeDtypeStruct((TOP_K, T), jnp.int32),
            jax.ShapeDtypeStruct((TOP_K, T), jnp.float32),
            jax.ShapeDtypeStruct((1, NB + 1), jnp.int32),
        ),
    )(x, Wr, expert_scale.reshape(1, NUM_EXPERTS))


def _dispatch_body(x_hbm, slot_hbm, xg_hbm, sbuf, rows, sem):
    c = lax.axis_index("c")
    s = lax.axis_index("s")
    wid = s * NC + c
    for ch in range(A_PER_W // DCH):
        a0 = wid * A_PER_W + ch * DCH
        r0 = lax.rem(a0, T)
        pltpu.sync_copy(x_hbm.at[pl.ds(r0, DCH)], rows)
        pltpu.sync_copy(slot_hbm.at[pl.ds(a0, DCH)], sbuf)
        pltpu.async_copy(rows, xg_hbm.at[sbuf], sem).wait()


@functools.cache
def _dispatch_kernel():
    return pl.kernel(
        _dispatch_body,
        out_type=jax.ShapeDtypeStruct((NPAD, HIDDEN), jnp.float32),
        mesh=plsc.VectorSubcoreMesh(core_axis_name="c", subcore_axis_name="s"),
        scratch_types=[
            pltpu.VMEM((DCH,), jnp.int32),
            pltpu.VMEM((DCH, HIDDEN), jnp.float32),
            pltpu.SemaphoreType.DMA,
        ],
    )


def _ffn_body(plan_ref, xg_ref, w1_ref, w2_ref, out_ref):
    b = pl.program_id(0)
    j = pl.program_id(1)
    na = plan_ref[0, NB]

    @pl.when(b < na)
    def _():
        h = jnp.dot(xg_ref[...], w1_ref[0], preferred_element_type=jnp.float32)
        h = h * jax.nn.sigmoid(h)
        y = jnp.dot(h, w2_ref[0], preferred_element_type=jnp.float32)

        @pl.when(j == 0)
        def _():
            out_ref[...] = y

        @pl.when(j > 0)
        def _():
            out_ref[...] += y


def _ffn(plan, xg, W1, W2):
    grid_spec = pltpu.PrefetchScalarGridSpec(
        num_scalar_prefetch=1,
        grid=(NB, NJ),
        in_specs=[
            pl.BlockSpec((BLK, HIDDEN),
                         lambda b, j, plan: (jnp.minimum(b, plan[0, NB] - 1), 0)),
            pl.BlockSpec((1, HIDDEN, JBLK), lambda b, j, plan: (plan[0, b], 0, j)),
            pl.BlockSpec((1, JBLK, HIDDEN), lambda b, j, plan: (plan[0, b], j, 0)),
        ],
        out_specs=pl.BlockSpec((BLK, HIDDEN), lambda b, j, plan: (b, 0)),
    )
    return pl.pallas_call(
        _ffn_body,
        grid_spec=grid_spec,
        out_shape=jax.ShapeDtypeStruct((NPAD, HIDDEN), jnp.float32),
    )(plan, xg, W1, W2)


def _combine_body(yg_hbm, slot_hbm, w_hbm, out_hbm, s0, s1, wa, wb, ra, rb, sem):
    c = lax.axis_index("c")
    s = lax.axis_index("s")
    wid = s * NC + c
    for ch in range(T_PER_W // CCH):
        t0 = wid * T_PER_W + ch * CCH
        pltpu.sync_copy(slot_hbm.at[pl.ds(t0, CCH)], s0)
        pltpu.sync_copy(slot_hbm.at[pl.ds(T + t0, CCH)], s1)
        pltpu.sync_copy(w_hbm.at[pl.ds(t0, CCH)], wa.at[pl.ds(0, CCH)])
        pltpu.sync_copy(w_hbm.at[pl.ds(T + t0, CCH)], wb.at[pl.ds(0, CCH)])
        cp0 = pltpu.async_copy(yg_hbm.at[s0], ra, sem)
        cp1 = pltpu.async_copy(yg_hbm.at[s1], rb, sem)
        cp0.wait()
        cp1.wait()

        def tok_body(i, _):
            wai = wa[pl.ds(i, 16)][0]
            wbi = wb[pl.ds(i, 16)][0]
            for v in range(HIDDEN // 16):
                off = v * 16
                ra[i, pl.ds(off, 16)] = (ra[i, pl.ds(off, 16)] * wai
                                         + rb[i, pl.ds(off, 16)] * wbi)
            return 0

        lax.fori_loop(0, CCH, tok_body, 0)
        pltpu.sync_copy(ra, out_hbm.at[pl.ds(t0, CCH)])


@functools.cache
def _combine_kernel():
    return pl.kernel(
        _combine_body,
        out_type=jax.ShapeDtypeStruct((T, HIDDEN), jnp.float32),
        mesh=plsc.VectorSubcoreMesh(core_axis_name="c", subcore_axis_name="s"),
        scratch_types=[
            pltpu.VMEM((CCH,), jnp.int32),
            pltpu.VMEM((CCH,), jnp.int32),
            pltpu.VMEM((CCH + 16,), jnp.float32),
            pltpu.VMEM((CCH + 16,), jnp.float32),
            pltpu.VMEM((CCH, HIDDEN), jnp.float32),
            pltpu.VMEM((CCH, HIDDEN), jnp.float32),
            pltpu.SemaphoreType.DMA,
        ],
    )


@jax.jit
def _moe(x, Wr, W1, W2, expert_scale):
    slot2, w2d, plan = _router(x, Wr, expert_scale)
    slot = slot2.reshape(TOP_K * T)
    wflat = w2d.reshape(TOP_K * T)
    xg = _dispatch_kernel()(x, slot)
    yg = _ffn(plan, xg, W1, W2)
    out = _combine_kernel()(yg, slot, wflat)
    return out


def kernel(hidden_states, Wr, W1, W2, expert_scale):
    b, s, d = hidden_states.shape
    x = hidden_states.reshape(b * s, d)
    out = _moe(x, Wr, W1, W2, expert_scale)
    return out.reshape(b, s, d)


# speedup vs baseline: 1.0212x; 1.0212x over previous
import functools

import jax
import jax.numpy as jnp
from jax import lax
from jax.experimental import pallas as pl
from jax.experimental.pallas import tpu as pltpu
from jax.experimental.pallas import tpu_sc as plsc

NUM_EXPERTS = 8
TOP_K = 2
HIDDEN = 1024
INTER = 4096
T = 2048

BLK = 768
JBLK = 2048
NJ = INTER // JBLK
NPAD = ((2 * T + NUM_EXPERTS * (BLK - 1)) // BLK + 1) * BLK
NB = NPAD // BLK

NC, NS = 2, 16
NW = NC * NS
A_PER_W = (TOP_K * T) // NW
T_PER_W = T // NW
DCH = 64
CCH = 16


def _cumsum_rows(x):
    n = x.shape[0]
    sh = 1
    while sh < n:
        x = x + jnp.concatenate(
            [jnp.zeros((sh,) + x.shape[1:], x.dtype), x[:-sh]], axis=0)
        sh *= 2
    return x


def _cumsum_lanes(x):
    n = x.shape[1]
    sh = 1
    while sh < n:
        x = x + jnp.concatenate(
            [jnp.zeros((x.shape[0], sh), x.dtype), x[:, :-sh]], axis=1)
        sh *= 2
    return x


def _router_body(x_ref, wr_ref, scale_ref, slot_ref, w_ref, plan_ref):
    x = x_ref[...]
    wr = wr_ref[...]
    logits = jnp.dot(x, wr, preferred_element_type=jnp.float32)
    m = jnp.max(logits, axis=-1, keepdims=True)
    p = jnp.exp(logits - m)
    p = p / jnp.sum(p, axis=-1, keepdims=True)

    e_iota = jax.lax.broadcasted_iota(jnp.int32, (T, NUM_EXPERTS), 1)

    p1 = jnp.max(p, axis=-1, keepdims=True)
    i1 = jnp.min(jnp.where(p == p1, e_iota, NUM_EXPERTS), axis=-1,
                 keepdims=True)
    masked = jnp.where(e_iota == i1, -jnp.inf, p)
    p2 = jnp.max(masked, axis=-1, keepdims=True)
    i2 = jnp.min(jnp.where(masked == p2, e_iota, NUM_EXPERTS), axis=-1,
                 keepdims=True)

    bb = jnp.exp(p2 - p1)
    w1 = 1.0 / (1.0 + bb)
    w2 = bb / (1.0 + bb)

    scale = scale_ref[...]
    s1 = jnp.sum(jnp.where(e_iota == i1, scale, 0.0), axis=-1, keepdims=True)
    s2 = jnp.sum(jnp.where(e_iota == i2, scale, 0.0), axis=-1, keepdims=True)

    oh0 = (e_iota == i1).astype(jnp.int32)
    oh1 = (e_iota == i2).astype(jnp.int32)
    c0 = _cumsum_rows(oh0)
    c1 = _cumsum_rows(oh1)
    pos0 = c0 - oh0
    pos1 = c1 - oh1
    count0 = c0[T - 1:T, :]
    counts = count0 + c1[T - 1:T, :]
    counts_pad = ((counts + (BLK - 1)) // BLK) * BLK
    base = _cumsum_lanes(counts_pad) - counts_pad

    dest0 = base + pos0
    dest1 = base + count0 + pos1
    slot0 = jnp.sum(oh0 * dest0, axis=-1)
    slot1 = jnp.sum(oh1 * dest1, axis=-1)
    slot_ref[0:1, :] = slot0.reshape(1, T)
    slot_ref[1:2, :] = slot1.reshape(1, T)
    w_ref[0:1, :] = (w1 * s1).reshape(1, T)
    w_ref[1:2, :] = (w2 * s2).reshape(1, T)

    ends = base + counts_pad
    b_iota = jax.lax.broadcasted_iota(jnp.int32, (NB, NUM_EXPERTS), 0) * BLK
    be = jnp.sum((b_iota >= ends).astype(jnp.int32), axis=-1)
    be = jnp.minimum(be, NUM_EXPERTS - 1)
    na = jnp.sum(counts_pad, axis=-1, keepdims=True) // BLK
    plan_ref[...] = jnp.concatenate([be.reshape(1, NB), na], axis=1)


def _router(x, Wr, expert_scale):
    return pl.pallas_call(
        _router_body,
        out_shape=(
            jax.ShapeDtypeStruct((TOP_K, T), jnp.int32),
            jax.ShapeDtypeStruct((TOP_K, T), jnp.float32),
            jax.ShapeDtypeStruct((1, NB + 1), jnp.int32),
        ),
    )(x, Wr, expert_scale.reshape(1, NUM_EXPERTS))


def _dispatch_body(x_hbm, slot_hbm, xg_hbm, sbuf, rows, sem):
    c = lax.axis_index("c")
    s = lax.axis_index("s")
    wid = s * NC + c
    for ch in range(A_PER_W // DCH):
        a0 = wid * A_PER_W + ch * DCH
        r0 = lax.rem(a0, T)
        pltpu.sync_copy(x_hbm.at[pl.ds(r0, DCH)], rows)
        pltpu.sync_copy(slot_hbm.at[pl.ds(a0, DCH)], sbuf)
        pltpu.async_copy(rows, xg_hbm.at[sbuf], sem).wait()


@functools.cache
def _dispatch_kernel():
    return pl.kernel(
        _dispatch_body,
        out_type=jax.ShapeDtypeStruct((NPAD, HIDDEN), jnp.float32),
        mesh=plsc.VectorSubcoreMesh(core_axis_name="c", subcore_axis_name="s"),
        scratch_types=[
            pltpu.VMEM((DCH,), jnp.int32),
            pltpu.VMEM((DCH, HIDDEN), jnp.float32),
            pltpu.SemaphoreType.DMA,
        ],
    )


def _ffn_body(plan_ref, xg_ref, w1_ref, w2_ref, out_ref):
    b = pl.program_id(0)
    j = pl.program_id(1)
    na = plan_ref[0, NB]

    @pl.when(b < na)
    def _():
        h = jnp.dot(xg_ref[...], w1_ref[0], preferred_element_type=jnp.float32)
        h = h * jax.nn.sigmoid(h)
        y = jnp.dot(h, w2_ref[0], preferred_element_type=jnp.float32)

        @pl.when(j == 0)
        def _():
            out_ref[...] = y

        @pl.when(j > 0)
        def _():
            out_ref[...] += y


def _ffn(plan, xg, W1, W2):
    grid_spec = pltpu.PrefetchScalarGridSpec(
        num_scalar_prefetch=1,
        grid=(NB, NJ),
        in_specs=[
            pl.BlockSpec((BLK, HIDDEN),
                         lambda b, j, plan: (jnp.minimum(b, plan[0, NB] - 1), 0)),
            pl.BlockSpec((1, HIDDEN, JBLK), lambda b, j, plan: (plan[0, b], 0, j)),
            pl.BlockSpec((1, JBLK, HIDDEN), lambda b, j, plan: (plan[0, b], j, 0)),
        ],
        out_specs=pl.BlockSpec((BLK, HIDDEN), lambda b, j, plan: (b, 0)),
    )
    return pl.pallas_call(
        _ffn_body,
        grid_spec=grid_spec,
        out_shape=jax.ShapeDtypeStruct((NPAD, HIDDEN), jnp.float32),
    )(plan, xg, W1, W2)


def _combine_body(yg_hbm, slot_hbm, w_hbm, out_hbm,
                  s0a, s1a, waa, wba, raa, rba, sema,
                  s0b, s1b, wab, wbb, rab, rbb, semb):
    c = lax.axis_index("c")
    s = lax.axis_index("s")
    wid = s * NC + c
    nch = T_PER_W // CCH
    bufs = [(s0a, s1a, waa, wba, raa, rba, sema),
            (s0b, s1b, wab, wbb, rab, rbb, semb)]

    def issue(ch, bset):
        s0, s1, wa, wb, ra, rb, sem = bset
        t0 = wid * T_PER_W + ch * CCH
        pltpu.sync_copy(slot_hbm.at[pl.ds(t0, CCH)], s0)
        pltpu.sync_copy(slot_hbm.at[pl.ds(T + t0, CCH)], s1)
        pltpu.sync_copy(w_hbm.at[pl.ds(t0, CCH)], wa.at[pl.ds(0, CCH)])
        pltpu.sync_copy(w_hbm.at[pl.ds(T + t0, CCH)], wb.at[pl.ds(0, CCH)])
        cp0 = pltpu.async_copy(yg_hbm.at[s0], ra, sem)
        cp1 = pltpu.async_copy(yg_hbm.at[s1], rb, sem)
        return cp0, cp1

    pend = issue(0, bufs[0])
    for ch in range(nch):
        nxt = issue(ch + 1, bufs[(ch + 1) % 2]) if ch + 1 < nch else None
        _, _, wa, wb, ra, rb, _ = bufs[ch % 2]
        pend[0].wait()
        pend[1].wait()

        def tok_body(i, _):
            wai = wa[pl.ds(i, 16)][0]
            wbi = wb[pl.ds(i, 16)][0]
            for v in range(HIDDEN // 16):
                off = v * 16
                ra[i, pl.ds(off, 16)] = (ra[i, pl.ds(off, 16)] * wai
                                         + rb[i, pl.ds(off, 16)] * wbi)
            return 0

        lax.fori_loop(0, CCH, tok_body, 0)
        t0 = wid * T_PER_W + ch * CCH
        pltpu.sync_copy(ra, out_hbm.at[pl.ds(t0, CCH)])
        pend = nxt


@functools.cache
def _combine_kernel():
    bufset = [
        pltpu.VMEM((CCH,), jnp.int32),
        pltpu.VMEM((CCH,), jnp.int32),
        pltpu.VMEM((CCH + 16,), jnp.float32),
        pltpu.VMEM((CCH + 16,), jnp.float32),
        pltpu.VMEM((CCH, HIDDEN), jnp.float32),
        pltpu.VMEM((CCH, HIDDEN), jnp.float32),
        pltpu.SemaphoreType.DMA,
    ]
    return pl.kernel(
        _combine_body,
        out_type=jax.ShapeDtypeStruct((T, HIDDEN), jnp.float32),
        mesh=plsc.VectorSubcoreMesh(core_axis_name="c", subcore_axis_name="s"),
        scratch_types=bufset + bufset,
    )


@jax.jit
def _moe(x, Wr, W1, W2, expert_scale):
    slot2, w2d, plan = _router(x, Wr, expert_scale)
    slot = slot2.reshape(TOP_K * T)
    wflat = w2d.reshape(TOP_K * T)
    xg = _dispatch_kernel()(x, slot)
    yg = _ffn(plan, xg, W1, W2)
    out = _combine_kernel()(yg, slot, wflat)
    return out


def kernel(hidden_states, Wr, W1, W2, expert_scale):
    b, s, d = hidden_states.shape
    x = hidden_states.reshape(b * s, d)
    out = _moe(x, Wr, W1, W2, expert_scale)
    return out.reshape(b, s, d)


# speedup vs baseline: 1.0231x; 1.0019x over previous
import functools

import jax
import jax.numpy as jnp
from jax import lax
from jax.experimental import pallas as pl
from jax.experimental.pallas import tpu as pltpu
from jax.experimental.pallas import tpu_sc as plsc

NUM_EXPERTS = 8
TOP_K = 2
HIDDEN = 1024
INTER = 4096
T = 2048

BLK = 768
JBLK = 2048
NJ = INTER // JBLK
NPAD = ((2 * T + NUM_EXPERTS * (BLK - 1)) // BLK + 1) * BLK
NB = NPAD // BLK

NC, NS = 2, 16
NW = NC * NS
A_PER_W = (TOP_K * T) // NW
T_PER_W = T // NW
DCH = 32
CCH = 16


def _cumsum_rows(x):
    n = x.shape[0]
    sh = 1
    while sh < n:
        x = x + jnp.concatenate(
            [jnp.zeros((sh,) + x.shape[1:], x.dtype), x[:-sh]], axis=0)
        sh *= 2
    return x


def _cumsum_lanes(x):
    n = x.shape[1]
    sh = 1
    while sh < n:
        x = x + jnp.concatenate(
            [jnp.zeros((x.shape[0], sh), x.dtype), x[:, :-sh]], axis=1)
        sh *= 2
    return x


def _router_body(x_ref, wr_ref, scale_ref, slot_ref, w_ref, plan_ref):
    x = x_ref[...]
    wr = wr_ref[...]
    logits = jnp.dot(x, wr, preferred_element_type=jnp.float32)
    m = jnp.max(logits, axis=-1, keepdims=True)
    p = jnp.exp(logits - m)
    p = p / jnp.sum(p, axis=-1, keepdims=True)

    e_iota = jax.lax.broadcasted_iota(jnp.int32, (T, NUM_EXPERTS), 1)

    p1 = jnp.max(p, axis=-1, keepdims=True)
    i1 = jnp.min(jnp.where(p == p1, e_iota, NUM_EXPERTS), axis=-1,
                 keepdims=True)
    masked = jnp.where(e_iota == i1, -jnp.inf, p)
    p2 = jnp.max(masked, axis=-1, keepdims=True)
    i2 = jnp.min(jnp.where(masked == p2, e_iota, NUM_EXPERTS), axis=-1,
                 keepdims=True)

    bb = jnp.exp(p2 - p1)
    w1 = 1.0 / (1.0 + bb)
    w2 = bb / (1.0 + bb)

    scale = scale_ref[...]
    s1 = jnp.sum(jnp.where(e_iota == i1, scale, 0.0), axis=-1, keepdims=True)
    s2 = jnp.sum(jnp.where(e_iota == i2, scale, 0.0), axis=-1, keepdims=True)

    oh0 = (e_iota == i1).astype(jnp.int32)
    oh1 = (e_iota == i2).astype(jnp.int32)
    c0 = _cumsum_rows(oh0)
    c1 = _cumsum_rows(oh1)
    pos0 = c0 - oh0
    pos1 = c1 - oh1
    count0 = c0[T - 1:T, :]
    counts = count0 + c1[T - 1:T, :]
    counts_pad = ((counts + (BLK - 1)) // BLK) * BLK
    base = _cumsum_lanes(counts_pad) - counts_pad

    dest0 = base + pos0
    dest1 = base + count0 + pos1
    slot0 = jnp.sum(oh0 * dest0, axis=-1)
    slot1 = jnp.sum(oh1 * dest1, axis=-1)
    slot_ref[0:1, :] = slot0.reshape(1, T)
    slot_ref[1:2, :] = slot1.reshape(1, T)
    w_ref[0:1, :] = (w1 * s1).reshape(1, T)
    w_ref[1:2, :] = (w2 * s2).reshape(1, T)

    ends = base + counts_pad
    b_iota = jax.lax.broadcasted_iota(jnp.int32, (NB, NUM_EXPERTS), 0) * BLK
    be = jnp.sum((b_iota >= ends).astype(jnp.int32), axis=-1)
    be = jnp.minimum(be, NUM_EXPERTS - 1)
    na = jnp.sum(counts_pad, axis=-1, keepdims=True) // BLK
    plan_ref[...] = jnp.concatenate([be.reshape(1, NB), na], axis=1)


def _router(x, Wr, expert_scale):
    return pl.pallas_call(
        _router_body,
        out_shape=(
            jax.ShapeDtypeStruct((TOP_K, T), jnp.int32),
            jax.ShapeDtypeStruct((TOP_K, T), jnp.float32),
            jax.ShapeDtypeStruct((1, NB + 1), jnp.int32),
        ),
    )(x, Wr, expert_scale.reshape(1, NUM_EXPERTS))


def _dispatch_body(x_hbm, slot_hbm, xg_hbm,
                   sbufa, rowsa, lsema, ssema,
                   sbufb, rowsb, lsemb, ssemb):
    c = lax.axis_index("c")
    s = lax.axis_index("s")
    wid = s * NC + c
    nch = A_PER_W // DCH
    bufs = [(sbufa, rowsa, lsema, ssema), (sbufb, rowsb, lsemb, ssemb)]

    def issue_load(ch, bset):
        sbuf, rows, lsem, _ = bset
        a0 = wid * A_PER_W + ch * DCH
        r0 = lax.rem(a0, T)
        pltpu.sync_copy(slot_hbm.at[pl.ds(a0, DCH)], sbuf)
        return pltpu.async_copy(x_hbm.at[pl.ds(r0, DCH)], rows, lsem)

    pend = issue_load(0, bufs[0])
    scat = None
    for ch in range(nch):
        nxt = issue_load(ch + 1, bufs[(ch + 1) % 2]) if ch + 1 < nch else None
        sbuf, rows, _, ssem = bufs[ch % 2]
        pend.wait()
        if scat is not None:
            scat.wait()
        scat = pltpu.async_copy(rows, xg_hbm.at[sbuf], ssem)
        pend = nxt
    scat.wait()


@functools.cache
def _dispatch_kernel():
    bufset = [
        pltpu.VMEM((DCH,), jnp.int32),
        pltpu.VMEM((DCH, HIDDEN), jnp.float32),
        pltpu.SemaphoreType.DMA,
        pltpu.SemaphoreType.DMA,
    ]
    return pl.kernel(
        _dispatch_body,
        out_type=jax.ShapeDtypeStruct((NPAD, HIDDEN), jnp.float32),
        mesh=plsc.VectorSubcoreMesh(core_axis_name="c", subcore_axis_name="s"),
        scratch_types=bufset + bufset,
    )


def _ffn_body(plan_ref, xg_ref, w1_ref, w2_ref, out_ref):
    b = pl.program_id(0)
    j = pl.program_id(1)
    na = plan_ref[0, NB]

    @pl.when(b < na)
    def _():
        h = jnp.dot(xg_ref[...], w1_ref[0], preferred_element_type=jnp.float32)
        h = h * jax.nn.sigmoid(h)
        y = jnp.dot(h, w2_ref[0], preferred_element_type=jnp.float32)

        @pl.when(j == 0)
        def _():
            out_ref[...] = y

        @pl.when(j > 0)
        def _():
            out_ref[...] += y


def _ffn(plan, xg, W1, W2):
    grid_spec = pltpu.PrefetchScalarGridSpec(
        num_scalar_prefetch=1,
        grid=(NB, NJ),
        in_specs=[
            pl.BlockSpec((BLK, HIDDEN),
                         lambda b, j, plan: (jnp.minimum(b, plan[0, NB] - 1), 0)),
            pl.BlockSpec((1, HIDDEN, JBLK), lambda b, j, plan: (plan[0, b], 0, j)),
            pl.BlockSpec((1, JBLK, HIDDEN), lambda b, j, plan: (plan[0, b], j, 0)),
        ],
        out_specs=pl.BlockSpec((BLK, HIDDEN), lambda b, j, plan: (b, 0)),
    )
    return pl.pallas_call(
        _ffn_body,
        grid_spec=grid_spec,
        out_shape=jax.ShapeDtypeStruct((NPAD, HIDDEN), jnp.float32),
    )(plan, xg, W1, W2)


def _combine_body(yg_hbm, slot_hbm, w_hbm, out_hbm,
                  s0a, s1a, waa, wba, raa, rba, sema,
                  s0b, s1b, wab, wbb, rab, rbb, semb):
    c = lax.axis_index("c")
    s = lax.axis_index("s")
    wid = s * NC + c
    nch = T_PER_W // CCH
    bufs = [(s0a, s1a, waa, wba, raa, rba, sema),
            (s0b, s1b, wab, wbb, rab, rbb, semb)]

    def issue(ch, bset):
        s0, s1, wa, wb, ra, rb, sem = bset
        t0 = wid * T_PER_W + ch * CCH
        pltpu.sync_copy(slot_hbm.at[pl.ds(t0, CCH)], s0)
        pltpu.sync_copy(slot_hbm.at[pl.ds(T + t0, CCH)], s1)
        pltpu.sync_copy(w_hbm.at[pl.ds(t0, CCH)], wa.at[pl.ds(0, CCH)])
        pltpu.sync_copy(w_hbm.at[pl.ds(T + t0, CCH)], wb.at[pl.ds(0, CCH)])
        cp0 = pltpu.async_copy(yg_hbm.at[s0], ra, sem)
        cp1 = pltpu.async_copy(yg_hbm.at[s1], rb, sem)
        return cp0, cp1

    pend = issue(0, bufs[0])
    for ch in range(nch):
        nxt = issue(ch + 1, bufs[(ch + 1) % 2]) if ch + 1 < nch else None
        _, _, wa, wb, ra, rb, _ = bufs[ch % 2]
        pend[0].wait()
        pend[1].wait()

        def tok_body(i, _):
            wai = wa[pl.ds(i, 16)][0]
            wbi = wb[pl.ds(i, 16)][0]
            for v in range(HIDDEN // 16):
                off = v * 16
                ra[i, pl.ds(off, 16)] = (ra[i, pl.ds(off, 16)] * wai
                                         + rb[i, pl.ds(off, 16)] * wbi)
            return 0

        lax.fori_loop(0, CCH, tok_body, 0)
        t0 = wid * T_PER_W + ch * CCH
        pltpu.sync_copy(ra, out_hbm.at[pl.ds(t0, CCH)])
        pend = nxt


@functools.cache
def _combine_kernel():
    bufset = [
        pltpu.VMEM((CCH,), jnp.int32),
        pltpu.VMEM((CCH,), jnp.int32),
        pltpu.VMEM((CCH + 16,), jnp.float32),
        pltpu.VMEM((CCH + 16,), jnp.float32),
        pltpu.VMEM((CCH, HIDDEN), jnp.float32),
        pltpu.VMEM((CCH, HIDDEN), jnp.float32),
        pltpu.SemaphoreType.DMA,
    ]
    return pl.kernel(
        _combine_body,
        out_type=jax.ShapeDtypeStruct((T, HIDDEN), jnp.float32),
        mesh=plsc.VectorSubcoreMesh(core_axis_name="c", subcore_axis_name="s"),
        scratch_types=bufset + bufset,
    )


@jax.jit
def _moe(x, Wr, W1, W2, expert_scale):
    slot2, w2d, plan = _router(x, Wr, expert_scale)
    slot = slot2.reshape(TOP_K * T)
    wflat = w2d.reshape(TOP_K * T)
    xg = _dispatch_kernel()(x, slot)
    yg = _ffn(plan, xg, W1, W2)
    out = _combine_kernel()(yg, slot, wflat)
    return out


def kernel(hidden_states, Wr, W1, W2, expert_scale):
    b, s, d = hidden_states.shape
    x = hidden_states.reshape(b * s, d)
    out = _moe(x, Wr, W1, W2, expert_scale)
    return out.reshape(b, s, d)
